# Initial kernel scaffold; baseline (speedup 1.0000x reference)
import jax
import jax.numpy as jnp
from jax.experimental import pallas as pl


def kernel(x, e, hyperedge_index, ln_gamma, ln_beta, W, b):
    raise NotImplementedError("write your pallas kernel here")


# speedup vs baseline: 4.6646x; 4.6646x over previous
import jax
import jax.numpy as jnp
from jax import lax
from jax.experimental import pallas as pl
from jax.experimental.pallas import tpu as pltpu
from jax.experimental.pallas import tpu_sc as plsc

_D = 6
_F = 256
_N = 10000
_NNZ = 160000
_EPS = 1e-5

_NW = 32
_CH = 256
_NCHUNK = _NNZ // _CH
_TMAX = (_NCHUNK + _NW - 1) // _NW
_OUTW = _CH * _D
_NVOUT = _OUTW // 16


def _table_body(x_ref, w_ref, o_ref):
    xb = x_ref[...]
    m = (xb[:, 0] + xb[:, 1] + xb[:, 2] + xb[:, 3] + xb[:, 4] + xb[:, 5]) * (1.0 / _D)
    p = jnp.dot(m, w_ref[...], preferred_element_type=jnp.float32)
    q = jnp.sum(m * m, axis=1, keepdims=True)
    col = lax.broadcasted_iota(jnp.int32, (1, 8), 1)
    o_ref[...] = p + q * (col == 7).astype(jnp.float32)


def _build_table(x3, w8, block=1000):
    n = x3.shape[0]
    return pl.pallas_call(
        _table_body,
        grid=(n // block,),
        in_specs=[
            pl.BlockSpec((block, _D, _F), lambda i: (i, 0, 0)),
            pl.BlockSpec((_F, 8), lambda i: (0, 0)),
        ],
        out_specs=pl.BlockSpec((block, 8), lambda i: (i, 0)),
        out_shape=jax.ShapeDtypeStruct((n, 8), jnp.float32),
    )(x3, w8)


def _pack_table(t):
    n = t.shape[0]
    p16 = t[:, :_D].astype(jnp.bfloat16).reshape(n, 3, 2)
    pw = lax.bitcast_convert_type(p16, jnp.int32)
    sq = lax.bitcast_convert_type(t[:, _D:_D + 2], jnp.int32)
    return jnp.concatenate([pw, sq], axis=1).reshape(-1)


def _rsqrt16(x):
    xi = plsc.bitcast(x, jnp.int32)
    yi = jnp.int32(0x5F3759DF) - lax.shift_right_logical(xi, 1)
    y = plsc.bitcast(yi, jnp.float32)
    for _ in range(3):
        y = y * (1.5 - 0.5 * x * y * y)
    return y


def _unpack6(w):
    out = []
    hi = jnp.int32(-65536)
    for k in range(3):
        out.append(plsc.bitcast(lax.shift_left(w[k], 16), jnp.float32))
        out.append(plsc.bitcast(w[k] & hi, jnp.float32))
    return out


def _sc_body(tx_hbm, te_hbm, row_hbm, col_hbm, par_hbm,
             idx_out, attr_out,
             tx_v, te_v, idx_r, idx_c, par_v,
             attr_buf, oidx_r, oidx_c):
    cid = lax.axis_index("c")
    sid = lax.axis_index("s")
    wid = sid * 2 + cid

    pltpu.sync_copy(tx_hbm, tx_v)
    pltpu.sync_copy(te_hbm, te_v)
    pltpu.sync_copy(par_hbm, par_v)
    iota = lax.broadcasted_iota(jnp.int32, (16,), 0)
    bb = [par_v[pl.ds(16 * j, 16)] for j in range(_D)]
    pat_i = [(16 * p + iota) // 6 for p in range(3)]
    pat_j = [(16 * p + iota) - 6 * pat_i[p] for p in range(3)]

    def chunk_body(t, carry):
        c = wid + _NW * t

        @pl.when(c < _NCHUNK)
        def _():
            base = c * _CH
            pltpu.sync_copy(row_hbm.at[pl.ds(base, _CH)], idx_r)
            pltpu.sync_copy(col_hbm.at[pl.ds(base, _CH)], idx_c)
            for g in range(_CH // 16):
                ir = idx_r[pl.ds(g * 16, 16)]
                ic = idx_c[pl.ds(g * 16, 16)]
                ir5 = ir * 5
                ic5 = ic * 5
                wx = [plsc.load_gather(tx_v, [ir5 + k]) for k in range(5)]
                we = [plsc.load_gather(te_v, [ic5 + k]) for k in range(5)]
                px = _unpack6(wx)
                pe = _unpack6(we)
                sx = plsc.bitcast(wx[3], jnp.float32)
                qx = plsc.bitcast(wx[4], jnp.float32)
                se = plsc.bitcast(we[3], jnp.float32)
                qe = plsc.bitcast(we[4], jnp.float32)
                mu = (sx + se) * (1.0 / (2 * _F))
                ms = (qx + qe) * (1.0 / (2 * _F))
                r = _rsqrt16(ms - mu * mu + _EPS)
                pos0 = (g * 16 + iota) * _D
                for j in range(_D):
                    z = (px[j] + pe[j]) * r + bb[j]
                    sig = 1.0 / (1.0 + jnp.exp(-z))
                    plsc.store_scatter(attr_buf, [pos0 + j], sig)
            for v in range(_NVOUT):
                p = v % 3
                ivec = pat_i[p] + 8 * (v // 3)
                vr = plsc.load_gather(idx_r, [ivec])
                vc = plsc.load_gather(idx_c, [ivec])
                oidx_r[pl.ds(16 * v, 16)] = vr * _D + pat_j[p]
                oidx_c[pl.ds(16 * v, 16)] = vc * _D + pat_j[p]
            obase = base * _D
            pltpu.sync_copy(attr_buf, attr_out.at[pl.ds(obase, _OUTW)])
            pltpu.sync_copy(oidx_r, idx_out.at[0, pl.ds(obase, _OUTW)])
            pltpu.sync_copy(oidx_c, idx_out.at[1, pl.ds(obase, _OUTW)])

        return carry

    lax.fori_loop(0, _TMAX, chunk_body, 0)


def _make_sc_call(interpret=False):
    return pl.kernel(
        _sc_body,
        out_type=(
            jax.ShapeDtypeStruct((2, _NNZ * _D), jnp.int32),
            jax.ShapeDtypeStruct((_NNZ * _D,), jnp.float32),
        ),
        mesh=plsc.VectorSubcoreMesh(
            core_axis_name="c", subcore_axis_name="s",
            num_cores=2, num_subcores=16),
        scratch_types=[
            pltpu.VMEM((_N * 5,), jnp.int32),
            pltpu.VMEM((_N * 5,), jnp.int32),
            pltpu.VMEM((_CH,), jnp.int32),
            pltpu.VMEM((_CH,), jnp.int32),
            pltpu.VMEM((16 * _D,), jnp.float32),
            pltpu.VMEM((_OUTW,), jnp.float32),
            pltpu.VMEM((_OUTW,), jnp.int32),
            pltpu.VMEM((_OUTW,), jnp.int32),
        ],
        compiler_params=pltpu.CompilerParams(needs_layout_passes=False),
        interpret=interpret,
    )


def kernel(x, e, hyperedge_index, ln_gamma, ln_beta, W, b):
    n = x.shape[0] // _D
    f = x.shape[1]
    Wg = ln_gamma[:, None] * W
    b2 = ln_beta @ W + b
    wsum = jnp.sum(Wg, axis=0)
    ones = jnp.ones((f, 1), jnp.float32)
    zeros = jnp.zeros((f, 1), jnp.float32)
    wx8 = jnp.concatenate([Wg[:f] - wsum[None, :] / (2 * f), ones, zeros], axis=1)
    we8 = jnp.concatenate([Wg[f:] - wsum[None, :] / (2 * f), ones, zeros], axis=1)
    params = jnp.repeat(b2, 16)

    tx = _pack_table(_build_table(x.reshape(n, _D, f), wx8))
    te = _pack_table(_build_table(e.reshape(n, _D, f), we8))

    row = hyperedge_index[0]
    col = hyperedge_index[1]
    hidx, attr = _make_sc_call()(tx, te, row, col, params)
    return hidx, attr


# speedup vs baseline: 8.5936x; 1.8423x over previous
import jax
import jax.numpy as jnp
from jax import lax
from jax.experimental import pallas as pl
from jax.experimental.pallas import tpu as pltpu
from jax.experimental.pallas import tpu_sc as plsc

_D = 6
_F = 256
_N = 10000
_NNZ = 160000
_EPS = 1e-5

_NW = 32
_CH = 256
_NCHUNK = _NNZ // _CH
_TMAX = (_NCHUNK + _NW - 1) // _NW
_OUTW = _CH * _D
_NVOUT = _OUTW // 16


def _bf16_rne(u):
    return lax.shift_right_logical(
        u + jnp.int32(0x7FFF) + (lax.shift_right_logical(u, 16) & 1), 16)


def _table_body(x_ref, w_ref, o_ref):
    xb = x_ref[...]
    x3 = xb.reshape(xb.shape[0] // _D, _D, _F)
    m = (x3[:, 0] + x3[:, 1] + x3[:, 2] + x3[:, 3] + x3[:, 4]
         + x3[:, 5]) * (1.0 / _D)
    p = jnp.dot(m, w_ref[...], preferred_element_type=jnp.float32)
    q = jnp.sum(m * m, axis=1, keepdims=True)
    col = lax.broadcasted_iota(jnp.int32, (1, 8), 1)
    t = p + q * (col == 7).astype(jnp.float32)
    u = lax.bitcast_convert_type(t, jnp.int32)
    bf = _bf16_rne(u)
    w01 = bf[:, 0:1] | lax.shift_left(bf[:, 1:2], 16)
    w23 = bf[:, 2:3] | lax.shift_left(bf[:, 3:4], 16)
    w45 = bf[:, 4:5] | lax.shift_left(bf[:, 5:6], 16)
    o_ref[...] = jnp.concatenate([w01, w23, w45, u[:, 6:7], u[:, 7:8]], axis=1)


def _build_table(x, w8, block=1000):
    n = x.shape[0] // _D
    return pl.pallas_call(
        _table_body,
        grid=(n // block,),
        in_specs=[
            pl.BlockSpec((block * _D, _F), lambda i: (i, 0)),
            pl.BlockSpec((_F, 8), lambda i: (0, 0)),
        ],
        out_specs=pl.BlockSpec((block, 5), lambda i: (i, 0)),
        out_shape=jax.ShapeDtypeStruct((n, 5), jnp.int32),
    )(x, w8).reshape(-1)


def _rsqrt16(x):
    xi = plsc.bitcast(x, jnp.int32)
    yi = jnp.int32(0x5F3759DF) - lax.shift_right_logical(xi, 1)
    y = plsc.bitcast(yi, jnp.float32)
    for _ in range(3):
        y = y * (1.5 - 0.5 * x * y * y)
    return y


def _unpack6(w):
    out = []
    hi = jnp.int32(-65536)
    for k in range(3):
        out.append(plsc.bitcast(lax.shift_left(w[k], 16), jnp.float32))
        out.append(plsc.bitcast(w[k] & hi, jnp.float32))
    return out


def _sc_body(tx_hbm, te_hbm, row_hbm, col_hbm, par_hbm,
             idx_out, attr_out,
             tx_v, te_v, idx_r, idx_c, par_v,
             attr_buf, oidx_r, oidx_c):
    cid = lax.axis_index("c")
    sid = lax.axis_index("s")
    wid = sid * 2 + cid

    pltpu.sync_copy(tx_hbm, tx_v)
    pltpu.sync_copy(te_hbm, te_v)
    pltpu.sync_copy(par_hbm, par_v)
    iota = lax.broadcasted_iota(jnp.int32, (16,), 0)
    bb = [par_v[pl.ds(16 * j, 16)] for j in range(_D)]
    pat_i = [(16 * p + iota) // 6 for p in range(3)]
    pat_j = [(16 * p + iota) - 6 * pat_i[p] for p in range(3)]

    def chunk_body(t, carry):
        c = wid + _NW * t

        @pl.when(c < _NCHUNK)
        def _():
            base = c * _CH
            pltpu.sync_copy(row_hbm.at[pl.ds(base, _CH)], idx_r)
            pltpu.sync_copy(col_hbm.at[pl.ds(base, _CH)], idx_c)
            for g in range(_CH // 16):
                ir = idx_r[pl.ds(g * 16, 16)]
                ic = idx_c[pl.ds(g * 16, 16)]
                ir5 = ir * 5
                ic5 = ic * 5
                wx = [plsc.load_gather(tx_v, [ir5 + k]) for k in range(5)]
                we = [plsc.load_gather(te_v, [ic5 + k]) for k in range(5)]
                px = _unpack6(wx)
                pe = _unpack6(we)
                sx = plsc.bitcast(wx[3], jnp.float32)
                qx = plsc.bitcast(wx[4], jnp.float32)
                se = plsc.bitcast(we[3], jnp.float32)
                qe = plsc.bitcast(we[4], jnp.float32)
                mu = (sx + se) * (1.0 / (2 * _F))
                ms = (qx + qe) * (1.0 / (2 * _F))
                r = _rsqrt16(ms - mu * mu + _EPS)
                pos0 = (g * 16 + iota) * _D
                for j in range(_D):
                    z = (px[j] + pe[j]) * r + bb[j]
                    sig = 1.0 / (1.0 + jnp.exp(-z))
                    plsc.store_scatter(attr_buf, [pos0 + j], sig)
            for v in range(_NVOUT):
                p = v % 3
                ivec = pat_i[p] + 8 * (v // 3)
                vr = plsc.load_gather(idx_r, [ivec])
                vc = plsc.load_gather(idx_c, [ivec])
                oidx_r[pl.ds(16 * v, 16)] = vr * _D + pat_j[p]
                oidx_c[pl.ds(16 * v, 16)] = vc * _D + pat_j[p]
            obase = base * _D
            pltpu.sync_copy(attr_buf, attr_out.at[pl.ds(obase, _OUTW)])
            pltpu.sync_copy(oidx_r, idx_out.at[0, pl.ds(obase, _OUTW)])
            pltpu.sync_copy(oidx_c, idx_out.at[1, pl.ds(obase, _OUTW)])

        return carry

    lax.fori_loop(0, _TMAX, chunk_body, 0)


def _make_sc_call(interpret=False):
    return pl.kernel(
        _sc_body,
        out_type=(
            jax.ShapeDtypeStruct((2, _NNZ * _D), jnp.int32),
            jax.ShapeDtypeStruct((_NNZ * _D,), jnp.float32),
        ),
        mesh=plsc.VectorSubcoreMesh(
            core_axis_name="c", subcore_axis_name="s",
            num_cores=2, num_subcores=16),
        scratch_types=[
            pltpu.VMEM((_N * 5,), jnp.int32),
            pltpu.VMEM((_N * 5,), jnp.int32),
            pltpu.VMEM((_CH,), jnp.int32),
            pltpu.VMEM((_CH,), jnp.int32),
            pltpu.VMEM((16 * _D,), jnp.float32),
            pltpu.VMEM((_OUTW,), jnp.float32),
            pltpu.VMEM((_OUTW,), jnp.int32),
            pltpu.VMEM((_OUTW,), jnp.int32),
        ],
        compiler_params=pltpu.CompilerParams(needs_layout_passes=False),
        interpret=interpret,
    )


def kernel(x, e, hyperedge_index, ln_gamma, ln_beta, W, b):
    n = x.shape[0] // _D
    f = x.shape[1]
    Wg = ln_gamma[:, None] * W
    b2 = ln_beta @ W + b
    wsum = jnp.sum(Wg, axis=0)
    ones = jnp.ones((f, 1), jnp.float32)
    zeros = jnp.zeros((f, 1), jnp.float32)
    wx8 = jnp.concatenate([Wg[:f] - wsum[None, :] / (2 * f), ones, zeros], axis=1)
    we8 = jnp.concatenate([Wg[f:] - wsum[None, :] / (2 * f), ones, zeros], axis=1)
    params = jnp.repeat(b2, 16)

    tx = _build_table(x, wx8)
    te = _build_table(e, we8)

    row = hyperedge_index[0]
    col = hyperedge_index[1]
    hidx, attr = _make_sc_call()(tx, te, row, col, params)
    return hidx, attr


# speedup vs baseline: 8.9862x; 1.0457x over previous
import jax
import jax.numpy as jnp
from jax import lax
from jax.experimental import pallas as pl
from jax.experimental.pallas import tpu as pltpu
from jax.experimental.pallas import tpu_sc as plsc

_D = 6
_F = 256
_N = 10000
_NNZ = 160000
_EPS = 1e-5

_NW = 32
_CH = 256
_NCHUNK = _NNZ // _CH
_TMAX = (_NCHUNK + _NW - 1) // _NW
_OUTW = _CH * _D
_NVOUT = _OUTW // 16


def _bf16_rne(u):
    return lax.shift_right_logical(
        u + jnp.int32(0x7FFF) + (lax.shift_right_logical(u, 16) & 1), 16)


def _table_body(x_ref, w_ref, o_ref):
    xb = x_ref[...]
    x3 = xb.reshape(xb.shape[0] // _D, _D, _F)
    m = (x3[:, 0] + x3[:, 1] + x3[:, 2] + x3[:, 3] + x3[:, 4]
         + x3[:, 5]) * (1.0 / _D)
    p = jnp.dot(m, w_ref[...], preferred_element_type=jnp.float32)
    q = jnp.sum(m * m, axis=1, keepdims=True)
    col = lax.broadcasted_iota(jnp.int32, (1, 8), 1)
    t = p + q * (col == 7).astype(jnp.float32)
    u = lax.bitcast_convert_type(t, jnp.int32)
    bf = _bf16_rne(u)
    w01 = bf[:, 0:1] | lax.shift_left(bf[:, 1:2], 16)
    w23 = bf[:, 2:3] | lax.shift_left(bf[:, 3:4], 16)
    w45 = bf[:, 4:5] | lax.shift_left(bf[:, 5:6], 16)
    o_ref[...] = jnp.concatenate([w01, w23, w45, u[:, 6:7], u[:, 7:8]], axis=1)


def _build_table(x, w8, block=1000):
    n = x.shape[0] // _D
    return pl.pallas_call(
        _table_body,
        grid=(n // block,),
        in_specs=[
            pl.BlockSpec((block * _D, _F), lambda i: (i, 0)),
            pl.BlockSpec((_F, 8), lambda i: (0, 0)),
        ],
        out_specs=pl.BlockSpec((block, 5), lambda i: (i, 0)),
        out_shape=jax.ShapeDtypeStruct((n, 5), jnp.int32),
    )(x, w8).reshape(-1)


def _rsqrt16(x):
    xi = plsc.bitcast(x, jnp.int32)
    yi = jnp.int32(0x5F3759DF) - lax.shift_right_logical(xi, 1)
    y = plsc.bitcast(yi, jnp.float32)
    for _ in range(3):
        y = y * (1.5 - 0.5 * x * y * y)
    return y


def _unpack6(w):
    out = []
    hi = jnp.int32(-65536)
    for k in range(3):
        out.append(plsc.bitcast(lax.shift_left(w[k], 16), jnp.float32))
        out.append(plsc.bitcast(w[k] & hi, jnp.float32))
    return out


def _sc_body(tx_hbm, te_hbm, row_hbm, col_hbm, par_hbm,
             idx_out, attr_out,
             tx_v, te_v, ir0, ir1, ic0, ic1, par_v,
             a0, a1, or0, or1, oc0, oc1,
             si0, si1, so0, so1):
    cid = lax.axis_index("c")
    sid = lax.axis_index("s")
    wid = sid * 2 + cid

    pltpu.sync_copy(tx_hbm, tx_v)
    pltpu.sync_copy(te_hbm, te_v)
    pltpu.sync_copy(par_hbm, par_v)
    iota = lax.broadcasted_iota(jnp.int32, (16,), 0)
    bb = [par_v[pl.ds(16 * j, 16)] for j in range(_D)]
    pat_i = [(16 * p + iota) // 6 for p in range(3)]
    pat_j = [(16 * p + iota) - 6 * pat_i[p] for p in range(3)]

    irs, ics = (ir0, ir1), (ic0, ic1)
    ats, ors, ocs = (a0, a1), (or0, or1), (oc0, oc1)
    sis, sos = (si0, si1), (so0, so1)

    def issue_in(t, b):
        c = wid + _NW * t

        @pl.when(c < _NCHUNK)
        def _():
            base = c * _CH
            pltpu.async_copy(row_hbm.at[pl.ds(base, _CH)], irs[b], sis[b])
            pltpu.async_copy(col_hbm.at[pl.ds(base, _CH)], ics[b], sis[b])

    def wait_in(b):
        pltpu.make_async_copy(row_hbm.at[pl.ds(0, _CH)], irs[b], sis[b]).wait()
        pltpu.make_async_copy(col_hbm.at[pl.ds(0, _CH)], ics[b], sis[b]).wait()

    def drain_out(b):
        pltpu.make_async_copy(ats[b], attr_out.at[pl.ds(0, _OUTW)], sos[b]).wait()
        pltpu.make_async_copy(ors[b], idx_out.at[0, pl.ds(0, _OUTW)], sos[b]).wait()
        pltpu.make_async_copy(ocs[b], idx_out.at[1, pl.ds(0, _OUTW)], sos[b]).wait()

    def compute(b):
        idx_r, idx_c = irs[b], ics[b]
        attr_buf, oidx_r, oidx_c = ats[b], ors[b], ocs[b]
        for g in range(_CH // 16):
            ir = idx_r[pl.ds(g * 16, 16)]
            ic = idx_c[pl.ds(g * 16, 16)]
            ir5 = ir * 5
            ic5 = ic * 5
            wx = [plsc.load_gather(tx_v, [ir5 + k]) for k in range(5)]
            we = [plsc.load_gather(te_v, [ic5 + k]) for k in range(5)]
            px = _unpack6(wx)
            pe = _unpack6(we)
            sx = plsc.bitcast(wx[3], jnp.float32)
            qx = plsc.bitcast(wx[4], jnp.float32)
            se = plsc.bitcast(we[3], jnp.float32)
            qe = plsc.bitcast(we[4], jnp.float32)
            mu = (sx + se) * (1.0 / (2 * _F))
            ms = (qx + qe) * (1.0 / (2 * _F))
            r = _rsqrt16(ms - mu * mu + _EPS)
            pos0 = (g * 16 + iota) * _D
            for j in range(_D):
                z = (px[j] + pe[j]) * r + bb[j]
                sig = 1.0 / (1.0 + jnp.exp(-z))
                plsc.store_scatter(attr_buf, [pos0 + j], sig)
        for v in range(_NVOUT):
            p = v % 3
            ivec = pat_i[p] + 8 * (v // 3)
            vr = plsc.load_gather(idx_r, [ivec])
            vc = plsc.load_gather(idx_c, [ivec])
            oidx_r[pl.ds(16 * v, 16)] = vr * _D + pat_j[p]
            oidx_c[pl.ds(16 * v, 16)] = vc * _D + pat_j[p]

    issue_in(0, 0)
    issue_in(1, 1)

    def outer(T, carry):
        for b in range(2):
            t = 2 * T + b
            c = wid + _NW * t

            @pl.when(c < _NCHUNK)
            def _(b=b, t=t, c=c):
                wait_in(b)

                @pl.when(t >= 2)
                def _():
                    drain_out(b)

                compute(b)
                base = c * _CH
                obase = base * _D
                pltpu.async_copy(ats[b], attr_out.at[pl.ds(obase, _OUTW)], sos[b])
                pltpu.async_copy(ors[b], idx_out.at[0, pl.ds(obase, _OUTW)], sos[b])
                pltpu.async_copy(ocs[b], idx_out.at[1, pl.ds(obase, _OUTW)], sos[b])
                issue_in(t + 2, b)

        return carry

    lax.fori_loop(0, _TMAX // 2, outer, 0)
    drain_out(0)
    drain_out(1)


def _make_sc_call(interpret=False):
    return pl.kernel(
        _sc_body,
        out_type=(
            jax.ShapeDtypeStruct((2, _NNZ * _D), jnp.int32),
            jax.ShapeDtypeStruct((_NNZ * _D,), jnp.float32),
        ),
        mesh=plsc.VectorSubcoreMesh(
            core_axis_name="c", subcore_axis_name="s",
            num_cores=2, num_subcores=16),
        scratch_types=[
            pltpu.VMEM((_N * 5,), jnp.int32),
            pltpu.VMEM((_N * 5,), jnp.int32),
            pltpu.VMEM((_CH,), jnp.int32),
            pltpu.VMEM((_CH,), jnp.int32),
            pltpu.VMEM((_CH,), jnp.int32),
            pltpu.VMEM((_CH,), jnp.int32),
            pltpu.VMEM((16 * _D,), jnp.float32),
            pltpu.VMEM((_OUTW,), jnp.float32),
            pltpu.VMEM((_OUTW,), jnp.float32),
            pltpu.VMEM((_OUTW,), jnp.int32),
            pltpu.VMEM((_OUTW,), jnp.int32),
            pltpu.VMEM((_OUTW,), jnp.int32),
            pltpu.VMEM((_OUTW,), jnp.int32),
            pltpu.SemaphoreType.DMA,
            pltpu.SemaphoreType.DMA,
            pltpu.SemaphoreType.DMA,
            pltpu.SemaphoreType.DMA,
        ],
        compiler_params=pltpu.CompilerParams(needs_layout_passes=False),
        interpret=interpret,
    )


def kernel(x, e, hyperedge_index, ln_gamma, ln_beta, W, b):
    n = x.shape[0] // _D
    f = x.shape[1]
    Wg = ln_gamma[:, None] * W
    b2 = ln_beta @ W + b
    wsum = jnp.sum(Wg, axis=0)
    ones = jnp.ones((f, 1), jnp.float32)
    zeros = jnp.zeros((f, 1), jnp.float32)
    wx8 = jnp.concatenate([Wg[:f] - wsum[None, :] / (2 * f), ones, zeros], axis=1)
    we8 = jnp.concatenate([Wg[f:] - wsum[None, :] / (2 * f), ones, zeros], axis=1)
    params = jnp.repeat(b2, 16)

    tx = _build_table(x, wx8)
    te = _build_table(e, we8)

    row = hyperedge_index[0]
    col = hyperedge_index[1]
    hidx, attr = _make_sc_call()(tx, te, row, col, params)
    return hidx, attr


# speedup vs baseline: 10.5411x; 1.1730x over previous
import jax
import jax.numpy as jnp
from jax import lax
from jax.experimental import pallas as pl
from jax.experimental.pallas import tpu as pltpu
from jax.experimental.pallas import tpu_sc as plsc

_D = 6
_F = 256
_N = 10000
_NNZ = 160000
_EPS = 1e-5

_NW = 32
_CH = 256
_NCHUNK = _NNZ // _CH
_TMAX = (_NCHUNK + _NW - 1) // _NW
_OUTW = _CH * _D
_NVOUT = _OUTW // 16


def _bf16_rne(u):
    return lax.shift_right_logical(
        u + jnp.int32(0x7FFF) + (lax.shift_right_logical(u, 16) & 1), 16)


def _table_body(x_ref, w_ref, o_ref):
    xb = x_ref[...]
    x3 = xb.reshape(xb.shape[0] // _D, _D, _F)
    m = (x3[:, 0] + x3[:, 1] + x3[:, 2] + x3[:, 3] + x3[:, 4]
         + x3[:, 5]) * (1.0 / _D)
    p = jnp.dot(m, w_ref[...], preferred_element_type=jnp.float32)
    q = jnp.sum(m * m, axis=1, keepdims=True)
    col = lax.broadcasted_iota(jnp.int32, (1, 8), 1)
    t = p + q * (col == 7).astype(jnp.float32)
    u = lax.bitcast_convert_type(t, jnp.int32)
    bf = _bf16_rne(u)
    w01 = bf[:, 0:1] | lax.shift_left(bf[:, 1:2], 16)
    w23 = bf[:, 2:3] | lax.shift_left(bf[:, 3:4], 16)
    w45 = bf[:, 4:5] | lax.shift_left(bf[:, 5:6], 16)
    o_ref[...] = jnp.concatenate([w01, w23, w45, u[:, 6:7], u[:, 7:8]], axis=1)


def _build_table(x, w8, block=1000):
    n = x.shape[0] // _D
    return pl.pallas_call(
        _table_body,
        grid=(n // block,),
        in_specs=[
            pl.BlockSpec((block * _D, _F), lambda i: (i, 0)),
            pl.BlockSpec((_F, 8), lambda i: (0, 0)),
        ],
        out_specs=pl.BlockSpec((block, 5), lambda i: (i, 0)),
        out_shape=jax.ShapeDtypeStruct((n, 5), jnp.int32),
    )(x, w8).reshape(-1)


def _hidx_body(x_ref, e_ref, j_ref, o_ref):
    idxf = x_ref[...].astype(jnp.float32)
    out = jnp.dot(idxf, e_ref[...], preferred_element_type=jnp.float32,
                  precision=lax.Precision.HIGHEST)
    o_ref[...] = (out + j_ref[...]).astype(jnp.int32)


def _build_hidx(rc2, block=2500):
    lidx = jnp.arange(128)
    col = jnp.arange(768)
    e6 = jnp.where(col[None, :] // _D == lidx[:, None], 6.0, 0.0).astype(jnp.float32)
    jmod = (col[None, :] % _D).astype(jnp.float32)
    n = rc2.shape[0]
    return pl.pallas_call(
        _hidx_body,
        grid=(n // block,),
        in_specs=[
            pl.BlockSpec((block, 128), lambda i: (i, 0)),
            pl.BlockSpec((128, 768), lambda i: (0, 0)),
            pl.BlockSpec((1, 768), lambda i: (0, 0)),
        ],
        out_specs=pl.BlockSpec((block, 768), lambda i: (i, 0)),
        out_shape=jax.ShapeDtypeStruct((n, 768), jnp.int32),
    )(rc2, e6, jmod)


def _rsqrt16(x):
    xi = plsc.bitcast(x, jnp.int32)
    yi = jnp.int32(0x5F3759DF) - lax.shift_right_logical(xi, 1)
    y = plsc.bitcast(yi, jnp.float32)
    for _ in range(3):
        y = y * (1.5 - 0.5 * x * y * y)
    return y


def _unpack6(w):
    out = []
    hi = jnp.int32(-65536)
    for k in range(3):
        out.append(plsc.bitcast(lax.shift_left(w[k], 16), jnp.float32))
        out.append(plsc.bitcast(w[k] & hi, jnp.float32))
    return out


def _sc_body(tx_hbm, te_hbm, row_hbm, col_hbm, par_hbm,
             attr_out,
             tx_v, te_v, ir0, ir1, ic0, ic1, par_v,
             a0, a1,
             si0, si1, so0, so1):
    cid = lax.axis_index("c")
    sid = lax.axis_index("s")
    wid = sid * 2 + cid

    pltpu.sync_copy(tx_hbm, tx_v)
    pltpu.sync_copy(te_hbm, te_v)
    pltpu.sync_copy(par_hbm, par_v)
    iota = lax.broadcasted_iota(jnp.int32, (16,), 0)
    bb = [par_v[pl.ds(16 * j, 16)] for j in range(_D)]
    irs, ics = (ir0, ir1), (ic0, ic1)
    ats = (a0, a1)
    sis, sos = (si0, si1), (so0, so1)

    def issue_in(t, b):
        c = wid + _NW * t

        @pl.when(c < _NCHUNK)
        def _():
            base = c * _CH
            pltpu.async_copy(row_hbm.at[pl.ds(base, _CH)], irs[b], sis[b])
            pltpu.async_copy(col_hbm.at[pl.ds(base, _CH)], ics[b], sis[b])

    def wait_in(b):
        pltpu.make_async_copy(row_hbm.at[pl.ds(0, _CH)], irs[b], sis[b]).wait()
        pltpu.make_async_copy(col_hbm.at[pl.ds(0, _CH)], ics[b], sis[b]).wait()

    def drain_out(b):
        pltpu.make_async_copy(ats[b], attr_out.at[pl.ds(0, _OUTW)], sos[b]).wait()

    def compute(b):
        idx_r, idx_c = irs[b], ics[b]
        attr_buf = ats[b]
        for g in range(_CH // 16):
            ir = idx_r[pl.ds(g * 16, 16)]
            ic = idx_c[pl.ds(g * 16, 16)]
            ir5 = ir * 5
            ic5 = ic * 5
            wx = [plsc.load_gather(tx_v, [ir5 + k]) for k in range(5)]
            we = [plsc.load_gather(te_v, [ic5 + k]) for k in range(5)]
            px = _unpack6(wx)
            pe = _unpack6(we)
            sx = plsc.bitcast(wx[3], jnp.float32)
            qx = plsc.bitcast(wx[4], jnp.float32)
            se = plsc.bitcast(we[3], jnp.float32)
            qe = plsc.bitcast(we[4], jnp.float32)
            mu = (sx + se) * (1.0 / (2 * _F))
            ms = (qx + qe) * (1.0 / (2 * _F))
            r = _rsqrt16(ms - mu * mu + _EPS)
            pos0 = (g * 16 + iota) * _D
            for j in range(_D):
                z = (px[j] + pe[j]) * r + bb[j]
                sig = 1.0 / (1.0 + jnp.exp(-z))
                plsc.store_scatter(attr_buf, [pos0 + j], sig)

    issue_in(0, 0)
    issue_in(1, 1)

    def outer(T, carry):
        for b in range(2):
            t = 2 * T + b
            c = wid + _NW * t

            @pl.when(c < _NCHUNK)
            def _(b=b, t=t, c=c):
                wait_in(b)

                @pl.when(t >= 2)
                def _():
                    drain_out(b)

                compute(b)
                base = c * _CH
                obase = base * _D
                pltpu.async_copy(ats[b], attr_out.at[pl.ds(obase, _OUTW)], sos[b])
                issue_in(t + 2, b)

        return carry

    lax.fori_loop(0, _TMAX // 2, outer, 0)
    drain_out(0)
    drain_out(1)


def _make_sc_call(interpret=False):
    return pl.kernel(
        _sc_body,
        out_type=jax.ShapeDtypeStruct((_NNZ * _D,), jnp.float32),
        mesh=plsc.VectorSubcoreMesh(
            core_axis_name="c", subcore_axis_name="s",
            num_cores=2, num_subcores=16),
        scratch_types=[
            pltpu.VMEM((_N * 5,), jnp.int32),
            pltpu.VMEM((_N * 5,), jnp.int32),
            pltpu.VMEM((_CH,), jnp.int32),
            pltpu.VMEM((_CH,), jnp.int32),
            pltpu.VMEM((_CH,), jnp.int32),
            pltpu.VMEM((_CH,), jnp.int32),
            pltpu.VMEM((16 * _D,), jnp.float32),
            pltpu.VMEM((_OUTW,), jnp.float32),
            pltpu.VMEM((_OUTW,), jnp.float32),
            pltpu.SemaphoreType.DMA,
            pltpu.SemaphoreType.DMA,
            pltpu.SemaphoreType.DMA,
            pltpu.SemaphoreType.DMA,
        ],
        compiler_params=pltpu.CompilerParams(needs_layout_passes=False),
        interpret=interpret,
    )


def kernel(x, e, hyperedge_index, ln_gamma, ln_beta, W, b):
    n = x.shape[0] // _D
    f = x.shape[1]
    Wg = ln_gamma[:, None] * W
    b2 = ln_beta @ W + b
    wsum = jnp.sum(Wg, axis=0)
    ones = jnp.ones((f, 1), jnp.float32)
    zeros = jnp.zeros((f, 1), jnp.float32)
    wx8 = jnp.concatenate([Wg[:f] - wsum[None, :] / (2 * f), ones, zeros], axis=1)
    we8 = jnp.concatenate([Wg[f:] - wsum[None, :] / (2 * f), ones, zeros], axis=1)
    params = jnp.repeat(b2, 16)

    tx = _build_table(x, wx8)
    te = _build_table(e, we8)

    row = hyperedge_index[0]
    col = hyperedge_index[1]
    hidx = _build_hidx(hyperedge_index.reshape(-1, 128)).reshape(2, _NNZ * _D)
    attr = _make_sc_call()(tx, te, row, col, params)
    return hidx, attr


# speedup vs baseline: 10.8195x; 1.0264x over previous
import jax
import jax.numpy as jnp
from jax import lax
from jax.experimental import pallas as pl
from jax.experimental.pallas import tpu as pltpu
from jax.experimental.pallas import tpu_sc as plsc

_D = 6
_F = 256
_N = 10000
_NNZ = 160000
_EPS = 1e-5

_NW = 32
_CH = 256
_NCHUNK = _NNZ // _CH
_TMAX = (_NCHUNK + _NW - 1) // _NW
_OUTW = _CH * _D
_NVOUT = _OUTW // 16


def _bf16_rne(u):
    return lax.shift_right_logical(
        u + jnp.int32(0x7FFF) + (lax.shift_right_logical(u, 16) & 1), 16)


def _table_body(x_ref, w_ref, o_ref):
    xb = x_ref[...]
    x3 = xb.reshape(xb.shape[0] // _D, _D, _F)
    m = (x3[:, 0] + x3[:, 1] + x3[:, 2] + x3[:, 3] + x3[:, 4]
         + x3[:, 5]) * (1.0 / _D)
    t = jnp.dot(jnp.concatenate([m, m * m], axis=1), w_ref[...],
                preferred_element_type=jnp.float32)
    u = lax.bitcast_convert_type(t, jnp.int32)
    bf = _bf16_rne(u)
    w01 = bf[:, 0:1] | lax.shift_left(bf[:, 1:2], 16)
    w23 = bf[:, 2:3] | lax.shift_left(bf[:, 3:4], 16)
    w45 = bf[:, 4:5] | lax.shift_left(bf[:, 5:6], 16)
    o_ref[...] = jnp.concatenate([w01, w23, w45, u[:, 6:7], u[:, 7:8]], axis=1)


def _build_table(x, w8, block=1000):
    n = x.shape[0] // _D
    return pl.pallas_call(
        _table_body,
        grid=(n // block,),
        in_specs=[
            pl.BlockSpec((block * _D, _F), lambda i: (i, 0)),
            pl.BlockSpec((2 * _F, 8), lambda i: (0, 0)),
        ],
        out_specs=pl.BlockSpec((block, 5), lambda i: (i, 0)),
        out_shape=jax.ShapeDtypeStruct((n, 5), jnp.int32),
    )(x, w8).reshape(-1)


def _hidx_body(x_ref, e_ref, j_ref, o_ref):
    idxf = x_ref[...].astype(jnp.float32)
    out = jnp.dot(idxf, e_ref[...], preferred_element_type=jnp.float32,
                  precision=lax.Precision.HIGHEST)
    o_ref[...] = (out + j_ref[...]).astype(jnp.int32)


def _build_hidx(rc2, block=2500):
    lidx = jnp.arange(128)
    col = jnp.arange(768)
    e6 = jnp.where(col[None, :] // _D == lidx[:, None], 6.0, 0.0).astype(jnp.float32)
    jmod = (col[None, :] % _D).astype(jnp.float32)
    n = rc2.shape[0]
    return pl.pallas_call(
        _hidx_body,
        grid=(n // block,),
        in_specs=[
            pl.BlockSpec((block, 128), lambda i: (i, 0)),
            pl.BlockSpec((128, 768), lambda i: (0, 0)),
            pl.BlockSpec((1, 768), lambda i: (0, 0)),
        ],
        out_specs=pl.BlockSpec((block, 768), lambda i: (i, 0)),
        out_shape=jax.ShapeDtypeStruct((n, 768), jnp.int32),
    )(rc2, e6, jmod)


def _rsqrt16(x):
    xi = plsc.bitcast(x, jnp.int32)
    yi = jnp.int32(0x5F3759DF) - lax.shift_right_logical(xi, 1)
    y = plsc.bitcast(yi, jnp.float32)
    for _ in range(3):
        y = y * (1.5 - 0.5 * x * y * y)
    return y


def _unpack6(w):
    out = []
    hi = jnp.int32(-65536)
    for k in range(3):
        out.append(plsc.bitcast(lax.shift_left(w[k], 16), jnp.float32))
        out.append(plsc.bitcast(w[k] & hi, jnp.float32))
    return out


def _sc_body(tx_hbm, te_hbm, row_hbm, col_hbm, par_hbm,
             attr_out,
             tx_v, te_v, ir0, ir1, ic0, ic1, par_v,
             a0, a1,
             si0, si1, so0, so1):
    cid = lax.axis_index("c")
    sid = lax.axis_index("s")
    wid = sid * 2 + cid

    pltpu.sync_copy(tx_hbm, tx_v)
    pltpu.sync_copy(te_hbm, te_v)
    pltpu.sync_copy(par_hbm, par_v)
    iota = lax.broadcasted_iota(jnp.int32, (16,), 0)
    bb = [par_v[pl.ds(16 * j, 16)] for j in range(_D)]
    irs, ics = (ir0, ir1), (ic0, ic1)
    ats = (a0, a1)
    sis, sos = (si0, si1), (so0, so1)

    def issue_in(t, b):
        c = wid + _NW * t

        @pl.when(c < _NCHUNK)
        def _():
            base = c * _CH
            pltpu.async_copy(row_hbm.at[pl.ds(base, _CH)], irs[b], sis[b])
            pltpu.async_copy(col_hbm.at[pl.ds(base, _CH)], ics[b], sis[b])

    def wait_in(b):
        pltpu.make_async_copy(row_hbm.at[pl.ds(0, _CH)], irs[b], sis[b]).wait()
        pltpu.make_async_copy(col_hbm.at[pl.ds(0, _CH)], ics[b], sis[b]).wait()

    def drain_out(b):
        pltpu.make_async_copy(ats[b], attr_out.at[pl.ds(0, _OUTW)], sos[b]).wait()

    def compute(b):
        idx_r, idx_c = irs[b], ics[b]
        attr_buf = ats[b]
        for g in range(_CH // 16):
            ir = idx_r[pl.ds(g * 16, 16)]
            ic = idx_c[pl.ds(g * 16, 16)]
            ir5 = ir * 5
            ic5 = ic * 5
            wx = [plsc.load_gather(tx_v, [ir5 + k]) for k in range(5)]
            we = [plsc.load_gather(te_v, [ic5 + k]) for k in range(5)]
            px = _unpack6(wx)
            pe = _unpack6(we)
            sx = plsc.bitcast(wx[3], jnp.float32)
            qx = plsc.bitcast(wx[4], jnp.float32)
            se = plsc.bitcast(we[3], jnp.float32)
            qe = plsc.bitcast(we[4], jnp.float32)
            mu = (sx + se) * (1.0 / (2 * _F))
            ms = (qx + qe) * (1.0 / (2 * _F))
            r = _rsqrt16(ms - mu * mu + _EPS)
            pos0 = (g * 16 + iota) * _D
            for j in range(_D):
                z = (px[j] + pe[j]) * r + bb[j]
                sig = 1.0 / (1.0 + jnp.exp(-z))
                plsc.store_scatter(attr_buf, [pos0 + j], sig)

    issue_in(0, 0)
    issue_in(1, 1)

    def outer(T, carry):
        for b in range(2):
            t = 2 * T + b
            c = wid + _NW * t

            @pl.when(c < _NCHUNK)
            def _(b=b, t=t, c=c):
                wait_in(b)

                @pl.when(t >= 2)
                def _():
                    drain_out(b)

                compute(b)
                base = c * _CH
                obase = base * _D
                pltpu.async_copy(ats[b], attr_out.at[pl.ds(obase, _OUTW)], sos[b])
                issue_in(t + 2, b)

        return carry

    lax.fori_loop(0, _TMAX // 2, outer, 0)
    drain_out(0)
    drain_out(1)


def _make_sc_call(interpret=False):
    return pl.kernel(
        _sc_body,
        out_type=jax.ShapeDtypeStruct((_NNZ * _D,), jnp.float32),
        mesh=plsc.VectorSubcoreMesh(
            core_axis_name="c", subcore_axis_name="s",
            num_cores=2, num_subcores=16),
        scratch_types=[
            pltpu.VMEM((_N * 5,), jnp.int32),
            pltpu.VMEM((_N * 5,), jnp.int32),
            pltpu.VMEM((_CH,), jnp.int32),
            pltpu.VMEM((_CH,), jnp.int32),
            pltpu.VMEM((_CH,), jnp.int32),
            pltpu.VMEM((_CH,), jnp.int32),
            pltpu.VMEM((16 * _D,), jnp.float32),
            pltpu.VMEM((_OUTW,), jnp.float32),
            pltpu.VMEM((_OUTW,), jnp.float32),
            pltpu.SemaphoreType.DMA,
            pltpu.SemaphoreType.DMA,
            pltpu.SemaphoreType.DMA,
            pltpu.SemaphoreType.DMA,
        ],
        compiler_params=pltpu.CompilerParams(needs_layout_passes=False),
        interpret=interpret,
    )


def kernel(x, e, hyperedge_index, ln_gamma, ln_beta, W, b):
    n = x.shape[0] // _D
    f = x.shape[1]
    Wg = ln_gamma[:, None] * W
    b2 = ln_beta @ W + b
    wsum = jnp.sum(Wg, axis=0)
    ones = jnp.ones((f, 1), jnp.float32)
    zeros = jnp.zeros((f, 1), jnp.float32)
    zcol = jnp.zeros((f, _D), jnp.float32)
    wx8 = jnp.concatenate([
        jnp.concatenate([Wg[:f] - wsum[None, :] / (2 * f), ones, zeros], axis=1),
        jnp.concatenate([zcol, zeros, ones], axis=1)], axis=0)
    we8 = jnp.concatenate([
        jnp.concatenate([Wg[f:] - wsum[None, :] / (2 * f), ones, zeros], axis=1),
        jnp.concatenate([zcol, zeros, ones], axis=1)], axis=0)
    params = jnp.repeat(b2, 16)

    tx = _build_table(x, wx8)
    te = _build_table(e, we8)

    row = hyperedge_index[0]
    col = hyperedge_index[1]
    hidx = _build_hidx(hyperedge_index.reshape(-1, 128)).reshape(2, _NNZ * _D)
    attr = _make_sc_call()(tx, te, row, col, params)
    return hidx, attr


# speedup vs baseline: 14.3535x; 1.3266x over previous
import jax
import jax.numpy as jnp
from jax import lax
from jax.experimental import pallas as pl
from jax.experimental.pallas import tpu as pltpu
from jax.experimental.pallas import tpu_sc as plsc

_D = 6
_F = 256
_N = 10000
_NNZ = 160000
_EPS = 1e-5

_NW = 32
_CH = 256
_NCHUNK = _NNZ // _CH
_TMAX = (_NCHUNK + _NW - 1) // _NW
_OUTW = _CH * _D
_NVOUT = _OUTW // 16


def _bf16_rne(u):
    return lax.shift_right_logical(
        u + jnp.int32(0x7FFF) + (lax.shift_right_logical(u, 16) & 1), 16)


def _table_body(x_ref, w_ref, o_ref):
    xb = x_ref[...]
    x3 = xb.reshape(xb.shape[0] // _D, _D, _F)
    m = (x3[:, 0] + x3[:, 1] + x3[:, 2] + x3[:, 3] + x3[:, 4]
         + x3[:, 5]) * (1.0 / _D)
    t = jnp.dot(jnp.concatenate([m, m * m], axis=1), w_ref[...],
                preferred_element_type=jnp.float32)
    u = lax.bitcast_convert_type(t, jnp.int32)
    bf = _bf16_rne(u)
    w01 = bf[:, 0:1] | lax.shift_left(bf[:, 1:2], 16)
    w23 = bf[:, 2:3] | lax.shift_left(bf[:, 3:4], 16)
    w45 = bf[:, 4:5] | lax.shift_left(bf[:, 5:6], 16)
    o_ref[...] = jnp.concatenate([w01, w23, w45, u[:, 6:7], u[:, 7:8]], axis=1)


def _build_table(x, w8, block=1000):
    n = x.shape[0] // _D
    return pl.pallas_call(
        _table_body,
        grid=(n // block,),
        in_specs=[
            pl.BlockSpec((block * _D, _F), lambda i: (i, 0)),
            pl.BlockSpec((2 * _F, 8), lambda i: (0, 0)),
        ],
        out_specs=pl.BlockSpec((block, 5), lambda i: (i, 0)),
        out_shape=jax.ShapeDtypeStruct((n, 5), jnp.int32),
    )(x, w8).reshape(-1)


def _hidx_body(x_ref, e_ref, j_ref, o_ref):
    idxf = x_ref[...].astype(jnp.float32)
    out = jnp.dot(idxf, e_ref[...], preferred_element_type=jnp.float32,
                  precision=lax.Precision.HIGHEST)
    o_ref[...] = (out + j_ref[...]).astype(jnp.int32)


def _build_hidx(rc2, block=2500):
    lidx = jnp.arange(128)
    col = jnp.arange(768)
    e6 = jnp.where(col[None, :] // _D == lidx[:, None], 6.0, 0.0).astype(jnp.float32)
    jmod = (col[None, :] % _D).astype(jnp.float32)
    n = rc2.shape[0]
    return pl.pallas_call(
        _hidx_body,
        grid=(n // block,),
        in_specs=[
            pl.BlockSpec((block, 128), lambda i: (i, 0)),
            pl.BlockSpec((128, 768), lambda i: (0, 0)),
            pl.BlockSpec((1, 768), lambda i: (0, 0)),
        ],
        out_specs=pl.BlockSpec((block, 768), lambda i: (i, 0)),
        out_shape=jax.ShapeDtypeStruct((n, 768), jnp.int32),
    )(rc2, e6, jmod)


def _rsqrt16(x):
    xi = plsc.bitcast(x, jnp.int32)
    yi = jnp.int32(0x5F3759DF) - lax.shift_right_logical(xi, 1)
    y = plsc.bitcast(yi, jnp.float32)
    for _ in range(3):
        y = y * (1.5 - 0.5 * x * y * y)
    return y


def _unpack6(w):
    out = []
    hi = jnp.int32(-65536)
    for k in range(3):
        out.append(plsc.bitcast(lax.shift_left(w[k], 16), jnp.float32))
        out.append(plsc.bitcast(w[k] & hi, jnp.float32))
    return out


def _sc_body(tx_hbm, te_hbm, row_hbm, col_hbm, par_hbm,
             attr_out,
             tx_v, te_v, ir0, ir1, ic0, ic1, par_v,
             a0, a1,
             si0, si1, so0, so1):
    cid = lax.axis_index("c")
    sid = lax.axis_index("s")
    wid = sid * 2 + cid

    pltpu.sync_copy(tx_hbm, tx_v)
    pltpu.sync_copy(te_hbm, te_v)
    pltpu.sync_copy(par_hbm, par_v)
    iota = lax.broadcasted_iota(jnp.int32, (16,), 0)
    bb = [par_v[pl.ds(16 * j, 16)] for j in range(_D)]
    irs, ics = (ir0, ir1), (ic0, ic1)
    ats = (a0, a1)
    sis, sos = (si0, si1), (so0, so1)

    def issue_in(t, b):
        c = wid + _NW * t

        @pl.when(c < _NCHUNK)
        def _():
            base = c * _CH
            pltpu.async_copy(row_hbm.at[pl.ds(base, _CH)], irs[b], sis[b])
            pltpu.async_copy(col_hbm.at[pl.ds(base, _CH)], ics[b], sis[b])

    def wait_in(b):
        pltpu.make_async_copy(row_hbm.at[pl.ds(0, _CH)], irs[b], sis[b]).wait()
        pltpu.make_async_copy(col_hbm.at[pl.ds(0, _CH)], ics[b], sis[b]).wait()

    def drain_out(b):
        pltpu.make_async_copy(ats[b], attr_out.at[pl.ds(0, _OUTW)], sos[b]).wait()

    def compute(b):
        idx_r, idx_c = irs[b], ics[b]
        attr_buf = ats[b]

        @plsc.parallel_loop(0, _CH, step=16, unroll=4)
        def _grp(i0):
            ir = idx_r[pl.ds(i0, 16)]
            ic = idx_c[pl.ds(i0, 16)]
            ir5 = ir * 5
            ic5 = ic * 5
            wx = [plsc.load_gather(tx_v, [ir5 + k]) for k in range(5)]
            we = [plsc.load_gather(te_v, [ic5 + k]) for k in range(5)]
            px = _unpack6(wx)
            pe = _unpack6(we)
            sx = plsc.bitcast(wx[3], jnp.float32)
            qx = plsc.bitcast(wx[4], jnp.float32)
            se = plsc.bitcast(we[3], jnp.float32)
            qe = plsc.bitcast(we[4], jnp.float32)
            mu = (sx + se) * (1.0 / (2 * _F))
            ms = (qx + qe) * (1.0 / (2 * _F))
            r = _rsqrt16(ms - mu * mu + _EPS)
            pos0 = (i0 + iota) * _D
            for j in range(_D):
                z = (px[j] + pe[j]) * r + bb[j]
                sig = 1.0 / (1.0 + jnp.exp(-z))
                plsc.store_scatter(attr_buf, [pos0 + j], sig)

    issue_in(0, 0)
    issue_in(1, 1)

    def outer(T, carry):
        for b in range(2):
            t = 2 * T + b
            c = wid + _NW * t

            @pl.when(c < _NCHUNK)
            def _(b=b, t=t, c=c):
                wait_in(b)

                @pl.when(t >= 2)
                def _():
                    drain_out(b)

                compute(b)
                base = c * _CH
                obase = base * _D
                pltpu.async_copy(ats[b], attr_out.at[pl.ds(obase, _OUTW)], sos[b])
                issue_in(t + 2, b)

        return carry

    lax.fori_loop(0, _TMAX // 2, outer, 0)
    drain_out(0)
    drain_out(1)


def _make_sc_call(interpret=False):
    return pl.kernel(
        _sc_body,
        out_type=jax.ShapeDtypeStruct((_NNZ * _D,), jnp.float32),
        mesh=plsc.VectorSubcoreMesh(
            core_axis_name="c", subcore_axis_name="s",
            num_cores=2, num_subcores=16),
        scratch_types=[
            pltpu.VMEM((_N * 5,), jnp.int32),
            pltpu.VMEM((_N * 5,), jnp.int32),
            pltpu.VMEM((_CH,), jnp.int32),
            pltpu.VMEM((_CH,), jnp.int32),
            pltpu.VMEM((_CH,), jnp.int32),
            pltpu.VMEM((_CH,), jnp.int32),
            pltpu.VMEM((16 * _D,), jnp.float32),
            pltpu.VMEM((_OUTW,), jnp.float32),
            pltpu.VMEM((_OUTW,), jnp.float32),
            pltpu.SemaphoreType.DMA,
            pltpu.SemaphoreType.DMA,
            pltpu.SemaphoreType.DMA,
            pltpu.SemaphoreType.DMA,
        ],
        compiler_params=pltpu.CompilerParams(needs_layout_passes=False),
        interpret=interpret,
    )


def kernel(x, e, hyperedge_index, ln_gamma, ln_beta, W, b):
    n = x.shape[0] // _D
    f = x.shape[1]
    Wg = ln_gamma[:, None] * W
    b2 = ln_beta @ W + b
    wsum = jnp.sum(Wg, axis=0)
    ones = jnp.ones((f, 1), jnp.float32)
    zeros = jnp.zeros((f, 1), jnp.float32)
    zcol = jnp.zeros((f, _D), jnp.float32)
    wx8 = jnp.concatenate([
        jnp.concatenate([Wg[:f] - wsum[None, :] / (2 * f), ones, zeros], axis=1),
        jnp.concatenate([zcol, zeros, ones], axis=1)], axis=0)
    we8 = jnp.concatenate([
        jnp.concatenate([Wg[f:] - wsum[None, :] / (2 * f), ones, zeros], axis=1),
        jnp.concatenate([zcol, zeros, ones], axis=1)], axis=0)
    params = jnp.repeat(b2, 16)

    tx = _build_table(x, wx8)
    te = _build_table(e, we8)

    row = hyperedge_index[0]
    col = hyperedge_index[1]
    hidx = _build_hidx(hyperedge_index.reshape(-1, 128)).reshape(2, _NNZ * _D)
    attr = _make_sc_call()(tx, te, row, col, params)
    return hidx, attr
